# baseline (speedup 1.0000x reference)
import functools

import jax
import jax.numpy as jnp
from jax.experimental import pallas as pl
from jax.experimental.pallas import tpu as pltpu


def _gate_kernel(nsplit, sub, *refs):
    xs = refs[:nsplit]
    w_ref = refs[nsplit]
    idx_ref, wgt_ref = refs[nsplit + 1], refs[nsplit + 2]
    for j, x_ref in enumerate(xs):
        logits = jnp.dot(x_ref[:, :], w_ref[:, :],
                         preferred_element_type=jnp.float32)
        e = logits.shape[-1]
        iota = jax.lax.broadcasted_iota(jnp.int32, logits.shape, 1)
        m1 = jnp.max(logits, axis=1, keepdims=True)
        i1 = jnp.min(jnp.where(logits == m1, iota, e), axis=1, keepdims=True)
        masked = jnp.where(iota == i1, -jnp.inf, logits)
        m2 = jnp.max(masked, axis=1, keepdims=True)
        i2 = jnp.min(jnp.where(masked == m2, iota, e), axis=1, keepdims=True)
        t = jnp.exp(m2 - m1)
        w1 = 1.0 / (1.0 + t)
        sl = pl.ds(j * sub, sub)
        idx_ref[sl, :] = jnp.concatenate([i1, i2], axis=1)
        wgt_ref[sl, :] = jnp.concatenate([w1, 1.0 - w1], axis=1)


@functools.partial(jax.jit, static_argnames=("tile", "nsplit"))
def _gate(xf, wt, tile, nsplit):
    n, d = xf.shape
    e = wt.shape[1]
    sub = tile // nsplit

    def x_spec(j):
        return pl.BlockSpec((sub, d), lambda i, j=j: (i * nsplit + j, 0))

    idx, wgt = pl.pallas_call(
        functools.partial(_gate_kernel, nsplit, sub),
        grid=(n // tile,),
        in_specs=[x_spec(j) for j in range(nsplit)]
        + [pl.BlockSpec((d, e), lambda i: (0, 0))],
        out_specs=[
            pl.BlockSpec((tile, 2), lambda i: (i, 0)),
            pl.BlockSpec((tile, 2), lambda i: (i, 0)),
        ],
        out_shape=[
            jax.ShapeDtypeStruct((n, 2), jnp.int32),
            jax.ShapeDtypeStruct((n, 2), jnp.float32),
        ],
        compiler_params=pltpu.CompilerParams(
            dimension_semantics=("arbitrary",),
        ),
    )(*([xf] * nsplit), wt)
    return idx, wgt


def kernel(x, w_gate):
    b, t, d = x.shape
    xf = x.reshape(b * t, d)
    wt = w_gate.T
    idx, wgt = _gate(xf, wt, tile=4096, nsplit=8)
    return idx.reshape(b, t, 2), wgt.reshape(b, t, 2)
